# baseline (speedup 1.0000x reference)
import functools

import jax
import jax.numpy as jnp
from jax import lax
from jax.experimental import pallas as pl
from jax.experimental.pallas import tpu as pltpu
from jax.experimental.pallas import tpu_sc as plsc

NUM_NODES = 100000
NODE_DIM = 128
GLOBAL_DIM = 128
BATCH_SIZE = 256
MAX_CUBES = 256
MAX_MOVES = 24
HIDDEN = 128
NEG = -1000000000.0

NSLOTS = BATCH_SIZE * MAX_CUBES
NC, NS = 2, 16
NW = NC * NS
NPAD = 100352
SCAN_CH = NPAD // NS
BPT = BATCH_SIZE // NW
RCH = 256
CLAMP = NUM_NODES - RCH


def _iota16():
    return lax.iota(jnp.int32, 16)


def _sread(ref, i):
    return plsc.load_gather(ref, [jnp.full((16,), i, jnp.int32)])[0]


@functools.cache
def _make_sc_compact():
    mesh = plsc.VectorSubcoreMesh(
        core_axis_name="c", subcore_axis_name="s",
        num_cores=NC, num_subcores=NS)

    @functools.partial(
        pl.kernel,
        out_type=(
            jax.ShapeDtypeStruct((NSLOTS, NODE_DIM), jnp.float32),
            jax.ShapeDtypeStruct((NW * 16,), jnp.int32),
        ),
        mesh=mesh,
        compiler_params=pltpu.CompilerParams(needs_layout_passes=False),
        scratch_types=[
            pltpu.VMEM((SCAN_CH + 16,), jnp.int32),
            pltpu.VMEM((BATCH_SIZE,), jnp.int32),
            pltpu.VMEM((NS, BATCH_SIZE), jnp.int32),
            pltpu.VMEM((BATCH_SIZE + 16,), jnp.int32),
            pltpu.VMEM((RCH, NODE_DIM), jnp.float32),
            pltpu.VMEM((RCH, NODE_DIM), jnp.float32),
            pltpu.VMEM((RCH,), jnp.int32),
            pltpu.VMEM((RCH,), jnp.int32),
            pltpu.VMEM((RCH,), jnp.int32),
            pltpu.VMEM((MAX_CUBES, NODE_DIM), jnp.float32),
            pltpu.VMEM((16,), jnp.int32),
            pltpu.VMEM_SHARED((NS, BATCH_SIZE), jnp.int32),
            pltpu.SemaphoreType.DMA,
            pltpu.SemaphoreType.DMA,
            pltpu.SemaphoreType.DMA,
            pltpu.SemaphoreType.DMA,
            pltpu.SemaphoreType.DMA,
        ],
    )
    def sc_compact(batch_hbm, m_hbm, table_hbm, xg_hbm, counts_hbm,
                   bt_ext, s2loc, s2all, s2f, rows_a, rows_b, m_a, m_b,
                   list_v, stage, cnt_v, s2_sp, rsem_a, rsem_b, msem_a,
                   msem_b, wsem):
        cid = lax.axis_index("c")
        sid = lax.axis_index("s")
        wid = cid * NS + sid
        iota = _iota16()
        sent = jnp.full((16,), NUM_NODES, jnp.int32)

        a_base = sid * SCAN_CH

        @pl.when(sid == 0)
        def _():
            bt_ext[pl.ds(0, 16)] = jnp.full((16,), -1, jnp.int32)

        @pl.when(sid > 0)
        def _():
            pltpu.sync_copy(batch_hbm.at[pl.ds(a_base - 16, 16)],
                            bt_ext.at[pl.ds(0, 16)])

        pltpu.sync_copy(batch_hbm.at[pl.ds(a_base, SCAN_CH)],
                        bt_ext.at[pl.ds(16, SCAN_CH)])
        for j in range(BATCH_SIZE // 16):
            s2loc[pl.ds(j * 16, 16)] = sent

        def scan_step(j, carry):
            cur = bt_ext[pl.ds(16 + j * 16, 16)]
            prv = plsc.load_gather(bt_ext, [iota + (15 + j * 16)])
            bnd = cur != prv
            gid = a_base + j * 16 + iota
            plsc.store_scatter(s2loc, [cur], gid, mask=bnd)
            return carry

        lax.fori_loop(0, SCAN_CH // 16, scan_step, 0)

        pltpu.sync_copy(s2loc, s2_sp.at[sid])
        plsc.subcore_barrier()
        pltpu.sync_copy(s2_sp, s2all)

        carry = sent
        for j in range(BATCH_SIZE // 16 - 1, -1, -1):
            acc = s2all[0, pl.ds(j * 16, 16)]

            def mrow(r, a):
                return jnp.minimum(a, s2all[r, pl.ds(j * 16, 16)])

            acc = lax.fori_loop(1, NS, mrow, acc)
            rev = lax.rev(acc, (0,))
            cmin = -plsc.cummax(-rev)
            suf = lax.rev(cmin, (0,))
            z = jnp.minimum(suf, carry)
            s2f[pl.ds(j * 16, 16)] = z
            carry = jnp.broadcast_to(z[0], (16,))
        s2f[pl.ds(BATCH_SIZE, 16)] = sent

        rbufs = (rows_a, rows_b)
        mbufs = (m_a, m_b)
        rsems = (rsem_a, rsem_b)
        msems = (msem_a, msem_b)

        def _chunk_descs(k, astart, p):
            lo = astart + k * RCH
            phys = pl.multiple_of(jnp.minimum(lo, jnp.int32(CLAMP)), 8)
            dr = pltpu.make_async_copy(table_hbm.at[pl.ds(phys, RCH)],
                                       rbufs[p], rsems[p])
            dm = pltpu.make_async_copy(m_hbm.at[pl.ds(phys, RCH)],
                                       mbufs[p], msems[p])
            return lo, phys, dr, dm

        def per_batch(bl, cnt_vec):
            b = wid * BPT + bl
            start = _sread(s2f, b)
            end = _sread(s2f, b + 1)
            astart = lax.bitwise_and(start, jnp.int32(-8))
            nch = (end - astart + (RCH - 1)) // RCH
            nch2 = ((nch + 1) // 2) * 2

            @pl.when(nch > 0)
            def _():
                for p in range(2):
                    _, _, dr, dm = _chunk_descs(jnp.int32(p), astart, p)
                    dr.start()
                    dm.start()

            @pl.when(bl > 0)
            def _():
                pltpu.make_async_copy(
                    stage, xg_hbm.at[pl.ds(b * MAX_CUBES, MAX_CUBES)],
                    wsem).wait()

            def half(k, p, cnt):
                lo, phys, dr, dm = _chunk_descs(k, astart, p)
                dr.wait()
                dm.wait()
                m_v = mbufs[p]
                rows_v = rbufs[p]
                for j in range(RCH // 16):
                    mv = m_v[pl.ds(j * 16, 16)]
                    gid = phys + j * 16 + iota
                    keep = ((mv > 0) & (gid >= start) & (gid < end)
                            & (gid >= lo) & (gid < lo + RCH))
                    ki = keep.astype(jnp.int32)
                    pc = plsc.cumsum(ki)
                    rank = cnt + pc - 1
                    keep2 = keep & (rank < MAX_CUBES)
                    rel = j * 16 + iota
                    plsc.store_compressed(list_v.at[pl.ds(j * 16, 16)],
                                          rel, mask=keep2)
                    n2 = jnp.sum(keep2.astype(jnp.int32))
                    base2 = jnp.minimum(cnt, jnp.int32(MAX_CUBES))

                    def cp(o, _):
                        src = _sread(list_v, j * 16 + o)
                        dst = base2 + o
                        for c in range(NODE_DIM // 16):
                            stage[dst, pl.ds(c * 16, 16)] = (
                                rows_v[src, pl.ds(c * 16, 16)])
                        return _

                    lax.fori_loop(0, n2, cp, 0)
                    cnt = cnt + jnp.sum(ki)
                nk = k + 2

                @pl.when(nk < nch2)
                def _():
                    _, _, dr2, dm2 = _chunk_descs(nk, astart, p)
                    dr2.start()
                    dm2.start()

                return cnt

            def pair(kp, cnt):
                cnt = half(2 * kp, 0, cnt)
                cnt = half(2 * kp + 1, 1, cnt)
                return cnt

            cnt = lax.fori_loop(0, nch2 // 2, pair, jnp.int32(0))
            pltpu.make_async_copy(
                stage, xg_hbm.at[pl.ds(b * MAX_CUBES, MAX_CUBES)],
                wsem).start()
            nb = jnp.minimum(cnt, jnp.int32(MAX_CUBES))
            return jnp.where(iota == bl, nb, cnt_vec)

        cnt_vec = lax.fori_loop(0, BPT, per_batch, jnp.zeros((16,), jnp.int32))
        pltpu.make_async_copy(
            stage, xg_hbm.at[pl.ds((wid * BPT + BPT - 1) * MAX_CUBES,
                                   MAX_CUBES)], wsem).wait()
        cnt_v[pl.ds(0, 16)] = cnt_vec
        pltpu.sync_copy(cnt_v, counts_hbm.at[pl.ds(wid * 16, 16)])

    return sc_compact


def _sc_compact(batch_pad, m_i32, table):
    return _make_sc_compact()(batch_pad, m_i32, table)


TB = 8


def _mlp_body(xg_ref, gf_ref, cnt_ref, w1n_ref, w1g_ref, b1_ref, w2_ref,
              b2_ref, mm_ref, out_ref):
    gf = gf_ref[...]
    gv = lax.dot_general(gf, w1g_ref[...], (((1,), (1,)), ((), ())),
                         preferred_element_type=jnp.float32) + b1_ref[...]
    x = xg_ref[...]
    h = lax.dot_general(x, w1n_ref[...], (((1,), (1,)), ((), ())),
                        preferred_element_type=jnp.float32)
    gvx = jnp.broadcast_to(gv.reshape(TB, 1, HIDDEN),
                           (TB, MAX_CUBES, HIDDEN)).reshape(TB * MAX_CUBES,
                                                            HIDDEN)
    h = jnp.maximum(h + gvx, 0.0)
    cnt = cnt_ref[0, 0, :]
    c_iota = lax.broadcasted_iota(jnp.int32, (MAX_MOVES, MAX_CUBES), 1)
    w2 = w2_ref[...]
    b2t = b2_ref[...]
    for bl in range(TB):
        hb = h[bl * MAX_CUBES:(bl + 1) * MAX_CUBES, :]
        st = lax.dot_general(w2, hb, (((1,), (1,)), ((), ())),
                             preferred_element_type=jnp.float32) + b2t
        mmt = jnp.transpose(mm_ref[bl].astype(jnp.int32), (1, 0))
        out_ref[bl] = jnp.where((c_iota < cnt[bl]) & (mmt > 0), st, NEG)


def _tc_mlp(xg, gf, counts3, w1n, w1g, b1, w2, b2t, mm):
    return pl.pallas_call(
        _mlp_body,
        grid=(BATCH_SIZE // TB,),
        in_specs=[
            pl.BlockSpec((TB * MAX_CUBES, NODE_DIM), lambda i: (i, 0)),
            pl.BlockSpec((TB, GLOBAL_DIM), lambda i: (i, 0)),
            pl.BlockSpec((1, 1, TB), lambda i: (i, 0, 0)),
            pl.BlockSpec((HIDDEN, NODE_DIM), lambda i: (0, 0)),
            pl.BlockSpec((HIDDEN, GLOBAL_DIM), lambda i: (0, 0)),
            pl.BlockSpec((1, HIDDEN), lambda i: (0, 0)),
            pl.BlockSpec((MAX_MOVES, HIDDEN), lambda i: (0, 0)),
            pl.BlockSpec((MAX_MOVES, 1), lambda i: (0, 0)),
            pl.BlockSpec((TB, MAX_CUBES, MAX_MOVES), lambda i: (i, 0, 0)),
        ],
        out_specs=pl.BlockSpec((TB, MAX_MOVES, MAX_CUBES), lambda i: (i, 0, 0)),
        out_shape=jax.ShapeDtypeStruct((BATCH_SIZE, MAX_MOVES, MAX_CUBES),
                                       jnp.float32),
    )(xg, gf, counts3, w1n, w1g, b1, w2, b2t, mm)


def kernel(node_features, global_features, cube_mask, batch, move_mask, W1, b1,
           W2, b2):
    batch_pad = jnp.pad(batch.astype(jnp.int32), (0, NPAD - NUM_NODES),
                        constant_values=BATCH_SIZE - 1)
    m_i32 = cube_mask.astype(jnp.int32)
    xg, counts_pad = _sc_compact(batch_pad, m_i32, node_features)
    counts3 = counts_pad.reshape(NW, 2, 8)[:, :1, :BPT]
    out_t = _tc_mlp(xg, global_features, counts3,
                    W1[:, :NODE_DIM], W1[:, NODE_DIM:], b1.reshape(1, HIDDEN),
                    W2, b2.reshape(MAX_MOVES, 1), move_mask)
    return out_t.swapaxes(1, 2).reshape(BATCH_SIZE, BATCH_SIZE * MAX_MOVES)


# speedup vs baseline: 1.0442x; 1.0442x over previous
import functools

import jax
import jax.numpy as jnp
from jax import lax
from jax.experimental import pallas as pl
from jax.experimental.pallas import tpu as pltpu
from jax.experimental.pallas import tpu_sc as plsc

NUM_NODES = 100000
NODE_DIM = 128
GLOBAL_DIM = 128
BATCH_SIZE = 256
MAX_CUBES = 256
MAX_MOVES = 24
HIDDEN = 128
NEG = -1000000000.0

NSLOTS = BATCH_SIZE * MAX_CUBES
NC, NS = 2, 16
NW = NC * NS
NPAD = 100352
SCAN_CH = NPAD // NS
BPT = BATCH_SIZE // NW
RCH = 256
CLAMP = NUM_NODES - RCH


def _iota16():
    return lax.iota(jnp.int32, 16)


def _sread(ref, i):
    return plsc.load_gather(ref, [jnp.full((16,), i, jnp.int32)])[0]


@functools.cache
def _make_sc_compact():
    mesh = plsc.VectorSubcoreMesh(
        core_axis_name="c", subcore_axis_name="s",
        num_cores=NC, num_subcores=NS)

    @functools.partial(
        pl.kernel,
        out_type=(
            jax.ShapeDtypeStruct((NSLOTS, NODE_DIM), jnp.float32),
            jax.ShapeDtypeStruct((NW * 16,), jnp.int32),
        ),
        mesh=mesh,
        compiler_params=pltpu.CompilerParams(needs_layout_passes=False),
        scratch_types=[
            pltpu.VMEM((SCAN_CH + 16,), jnp.int32),
            pltpu.VMEM((BATCH_SIZE,), jnp.int32),
            pltpu.VMEM((NS, BATCH_SIZE), jnp.int32),
            pltpu.VMEM((BATCH_SIZE + 16,), jnp.int32),
            pltpu.VMEM((RCH, NODE_DIM), jnp.float32),
            pltpu.VMEM((RCH, NODE_DIM), jnp.float32),
            pltpu.VMEM((RCH,), jnp.int32),
            pltpu.VMEM((RCH,), jnp.int32),
            pltpu.VMEM((RCH,), jnp.int32),
            pltpu.VMEM((MAX_CUBES, NODE_DIM), jnp.float32),
            pltpu.VMEM((16,), jnp.int32),
            pltpu.VMEM_SHARED((NS, BATCH_SIZE), jnp.int32),
            pltpu.SemaphoreType.DMA,
            pltpu.SemaphoreType.DMA,
            pltpu.SemaphoreType.DMA,
            pltpu.SemaphoreType.DMA,
            pltpu.SemaphoreType.DMA,
        ],
    )
    def sc_compact(batch_hbm, m_hbm, table_hbm, xg_hbm, counts_hbm,
                   bt_ext, s2loc, s2all, s2f, rows_a, rows_b, m_a, m_b,
                   list_v, stage, cnt_v, s2_sp, rsem_a, rsem_b, msem_a,
                   msem_b, wsem):
        cid = lax.axis_index("c")
        sid = lax.axis_index("s")
        wid = cid * NS + sid
        iota = _iota16()
        sent = jnp.full((16,), NUM_NODES, jnp.int32)

        a_base = sid * SCAN_CH

        @pl.when(sid == 0)
        def _():
            bt_ext[pl.ds(0, 16)] = jnp.full((16,), -1, jnp.int32)

        @pl.when(sid > 0)
        def _():
            pltpu.sync_copy(batch_hbm.at[pl.ds(a_base - 16, 16)],
                            bt_ext.at[pl.ds(0, 16)])

        pltpu.sync_copy(batch_hbm.at[pl.ds(a_base, SCAN_CH)],
                        bt_ext.at[pl.ds(16, SCAN_CH)])
        for j in range(BATCH_SIZE // 16):
            s2loc[pl.ds(j * 16, 16)] = sent

        def scan_step(j, carry):
            cur = bt_ext[pl.ds(16 + j * 16, 16)]
            prv = plsc.load_gather(bt_ext, [iota + (15 + j * 16)])
            bnd = cur != prv
            gid = a_base + j * 16 + iota
            plsc.store_scatter(s2loc, [cur], gid, mask=bnd)
            return carry

        lax.fori_loop(0, SCAN_CH // 16, scan_step, 0)

        pltpu.sync_copy(s2loc, s2_sp.at[sid])
        plsc.subcore_barrier()
        pltpu.sync_copy(s2_sp, s2all)

        carry = sent
        for j in range(BATCH_SIZE // 16 - 1, -1, -1):
            acc = s2all[0, pl.ds(j * 16, 16)]

            def mrow(r, a):
                return jnp.minimum(a, s2all[r, pl.ds(j * 16, 16)])

            acc = lax.fori_loop(1, NS, mrow, acc)
            rev = lax.rev(acc, (0,))
            cmin = -plsc.cummax(-rev)
            suf = lax.rev(cmin, (0,))
            z = jnp.minimum(suf, carry)
            s2f[pl.ds(j * 16, 16)] = z
            carry = jnp.broadcast_to(z[0], (16,))
        s2f[pl.ds(BATCH_SIZE, 16)] = sent

        rbufs = (rows_a, rows_b)
        mbufs = (m_a, m_b)
        rsems = (rsem_a, rsem_b)
        msems = (msem_a, msem_b)

        def _chunk_descs(k, astart, p):
            lo = astart + k * RCH
            phys = pl.multiple_of(jnp.minimum(lo, jnp.int32(CLAMP)), 8)
            dr = pltpu.make_async_copy(table_hbm.at[pl.ds(phys, RCH)],
                                       rbufs[p], rsems[p])
            dm = pltpu.make_async_copy(m_hbm.at[pl.ds(phys, RCH)],
                                       mbufs[p], msems[p])
            return lo, phys, dr, dm

        def per_batch(bl, cnt_vec):
            b = wid * BPT + bl
            start = _sread(s2f, b)
            end = _sread(s2f, b + 1)
            astart = lax.bitwise_and(start, jnp.int32(-8))
            nch = (end - astart + (RCH - 1)) // RCH
            nch2 = ((nch + 1) // 2) * 2

            @pl.when(nch > 0)
            def _():
                for p in range(2):
                    _, _, dr, dm = _chunk_descs(jnp.int32(p), astart, p)
                    dr.start()
                    dm.start()

            @pl.when(bl > 0)
            def _():
                pltpu.make_async_copy(
                    stage, xg_hbm.at[pl.ds(b * MAX_CUBES, MAX_CUBES)],
                    wsem).wait()

            def half(k, p, cnt):
                lo, phys, dr, dm = _chunk_descs(k, astart, p)
                dr.wait()
                dm.wait()
                m_v = mbufs[p]
                rows_v = rbufs[p]
                for j in range(RCH // 16):
                    mv = m_v[pl.ds(j * 16, 16)]
                    gid = phys + j * 16 + iota
                    keep = ((mv > 0) & (gid >= start) & (gid < end)
                            & (gid >= lo) & (gid < lo + RCH))
                    ki = keep.astype(jnp.int32)
                    pc = plsc.cumsum(ki)
                    rank = cnt + pc - 1
                    keep2 = keep & (rank < MAX_CUBES)
                    rel = j * 16 + iota
                    plsc.store_compressed(list_v.at[pl.ds(j * 16, 16)],
                                          rel, mask=keep2)
                    n2 = jnp.sum(keep2.astype(jnp.int32))
                    base2 = jnp.minimum(cnt, jnp.int32(MAX_CUBES))

                    def cp(o, _):
                        src = _sread(list_v, j * 16 + o)
                        dst = base2 + o
                        for c in range(NODE_DIM // 16):
                            stage[dst, pl.ds(c * 16, 16)] = (
                                rows_v[src, pl.ds(c * 16, 16)])
                        return _

                    lax.fori_loop(0, n2, cp, 0)
                    cnt = cnt + jnp.sum(ki)
                nk = k + 2

                @pl.when(nk < nch2)
                def _():
                    _, _, dr2, dm2 = _chunk_descs(nk, astart, p)
                    dr2.start()
                    dm2.start()

                return cnt

            def pair(kp, cnt):
                cnt = half(2 * kp, 0, cnt)
                cnt = half(2 * kp + 1, 1, cnt)
                return cnt

            cnt = lax.fori_loop(0, nch2 // 2, pair, jnp.int32(0))
            pltpu.make_async_copy(
                stage, xg_hbm.at[pl.ds(b * MAX_CUBES, MAX_CUBES)],
                wsem).start()
            nb = jnp.minimum(cnt, jnp.int32(MAX_CUBES))
            return jnp.where(iota == bl, nb, cnt_vec)

        cnt_vec = lax.fori_loop(0, BPT, per_batch, jnp.zeros((16,), jnp.int32))
        pltpu.make_async_copy(
            stage, xg_hbm.at[pl.ds((wid * BPT + BPT - 1) * MAX_CUBES,
                                   MAX_CUBES)], wsem).wait()
        cnt_v[pl.ds(0, 16)] = cnt_vec
        pltpu.sync_copy(cnt_v, counts_hbm.at[pl.ds(wid * 16, 16)])

    return sc_compact


def _sc_compact(batch_pad, m_i32, table):
    return _make_sc_compact()(batch_pad, m_i32, table)


TB = 8


def _mlp_body(xg_ref, gf_ref, cnt_ref, w1n_ref, w1g_ref, b1_ref, w2_ref,
              b2_ref, mm_ref, out_ref):
    gf = gf_ref[...]
    gv = lax.dot_general(gf, w1g_ref[...], (((1,), (1,)), ((), ())),
                         preferred_element_type=jnp.float32) + b1_ref[...]
    x = xg_ref[...]
    h = lax.dot_general(x, w1n_ref[...], (((1,), (1,)), ((), ())),
                        preferred_element_type=jnp.float32)
    gvx = jnp.broadcast_to(gv.reshape(TB, 1, HIDDEN),
                           (TB, MAX_CUBES, HIDDEN)).reshape(TB * MAX_CUBES,
                                                            HIDDEN)
    h = jnp.maximum(h + gvx, 0.0)
    cnt = cnt_ref[0, 0, :]
    c_iota = lax.broadcasted_iota(jnp.int32, (MAX_MOVES, MAX_CUBES), 1)
    w2 = w2_ref[...]
    b2t = b2_ref[...]
    for bl in range(TB):
        hb = h[bl * MAX_CUBES:(bl + 1) * MAX_CUBES, :]
        st = lax.dot_general(w2, hb, (((1,), (1,)), ((), ())),
                             preferred_element_type=jnp.float32) + b2t
        out_ref[bl] = jnp.where((c_iota < cnt[bl]) & mm_ref[bl], st, NEG)


def _tc_mlp(xg, gf, counts3, w1n, w1g, b1, w2, b2t, mm):
    return pl.pallas_call(
        _mlp_body,
        grid=(BATCH_SIZE // TB,),
        in_specs=[
            pl.BlockSpec((TB * MAX_CUBES, NODE_DIM), lambda i: (i, 0)),
            pl.BlockSpec((TB, GLOBAL_DIM), lambda i: (i, 0)),
            pl.BlockSpec((1, 1, TB), lambda i: (i, 0, 0)),
            pl.BlockSpec((HIDDEN, NODE_DIM), lambda i: (0, 0)),
            pl.BlockSpec((HIDDEN, GLOBAL_DIM), lambda i: (0, 0)),
            pl.BlockSpec((1, HIDDEN), lambda i: (0, 0)),
            pl.BlockSpec((MAX_MOVES, HIDDEN), lambda i: (0, 0)),
            pl.BlockSpec((MAX_MOVES, 1), lambda i: (0, 0)),
            pl.BlockSpec((TB, MAX_MOVES, MAX_CUBES), lambda i: (i, 0, 0)),
        ],
        out_specs=pl.BlockSpec((TB, MAX_MOVES, MAX_CUBES), lambda i: (i, 0, 0)),
        out_shape=jax.ShapeDtypeStruct((BATCH_SIZE, MAX_MOVES, MAX_CUBES),
                                       jnp.float32),
    )(xg, gf, counts3, w1n, w1g, b1, w2, b2t, mm)


def kernel(node_features, global_features, cube_mask, batch, move_mask, W1, b1,
           W2, b2):
    batch_pad = jnp.pad(batch.astype(jnp.int32), (0, NPAD - NUM_NODES),
                        constant_values=BATCH_SIZE - 1)
    m_i32 = cube_mask.astype(jnp.int32)
    xg, counts_pad = _sc_compact(batch_pad, m_i32, node_features)
    counts3 = counts_pad.reshape(NW, 2, 8)[:, :1, :BPT]
    out_t = _tc_mlp(xg, global_features, counts3,
                    W1[:, :NODE_DIM], W1[:, NODE_DIM:], b1.reshape(1, HIDDEN),
                    W2, b2.reshape(MAX_MOVES, 1), move_mask.swapaxes(1, 2))
    return out_t.swapaxes(1, 2).reshape(BATCH_SIZE, BATCH_SIZE * MAX_MOVES)
